# Initial kernel scaffold; baseline (speedup 1.0000x reference)
import jax
import jax.numpy as jnp
from jax.experimental import pallas as pl


def kernel(x):
    raise NotImplementedError("write your pallas kernel here")


# speedup vs baseline: 1.0048x; 1.0048x over previous
import jax
import jax.numpy as jnp
from jax.experimental import pallas as pl
from jax.experimental.pallas import tpu as pltpu


def _bmax_kernel(x_ref, o_ref):
    o_ref[...] = jnp.max(x_ref[...], axis=1)


def _lane_dense(n_keep, feat):
    plane = n_keep * feat
    for cand_l in (2048, 1024, 512, 256, 128):
        if plane % cand_l == 0 and (plane // cand_l) % 8 == 0:
            return plane // cand_l, cand_l
    for cand_l in (2048, 1024, 512, 256, 128):
        if plane % cand_l == 0:
            return plane // cand_l, cand_l
    return n_keep, feat


def kernel(x):
    bs, n_red, n_keep, feat = x.shape
    itemsize = jnp.dtype(x.dtype).itemsize

    s_dim, l_dim = _lane_dense(n_keep, feat)
    x3 = x.reshape(bs, n_red, s_dim, l_dim)

    in_block = n_red * s_dim * l_dim * itemsize
    vmem_limit = int(min(2 * in_block + (4 << 20), 100 << 20))

    y3 = pl.pallas_call(
        _bmax_kernel,
        out_shape=jax.ShapeDtypeStruct((bs, s_dim, l_dim), x.dtype),
        grid=(bs,),
        in_specs=[
            pl.BlockSpec((1, n_red, s_dim, l_dim), lambda b: (b, 0, 0, 0)),
        ],
        out_specs=pl.BlockSpec((1, s_dim, l_dim), lambda b: (b, 0, 0)),
        compiler_params=pltpu.CompilerParams(
            dimension_semantics=("parallel",),
            vmem_limit_bytes=vmem_limit,
        ),
    )(x3)

    return y3.reshape(bs, n_keep, feat)
